# baseline (speedup 1.0000x reference)
import functools

import jax
import jax.numpy as jnp
from jax import lax
from jax.experimental import pallas as pl
from jax.experimental.pallas import tpu as pltpu
from jax.experimental.pallas import tpu_sc as plsc

N = 50000
E = 800000
G = 512
NPAD = 65536
ND = 51200
EPAD = 819200
EPT = EPAD // 16

_MESH = plsc.VectorSubcoreMesh(core_axis_name="c", subcore_axis_name="s")


def _fast_rsqrt(d):
    bits = lax.bitcast_convert_type(d, jnp.int32)
    y = lax.bitcast_convert_type(jnp.int32(0x5F3759DF) - (bits >> 1), jnp.float32)
    for _ in range(3):
        y = y * (1.5 - 0.5 * d * y * y)
    return y


def _deg_body(dst_hbm, dinv_hbm, degspm, zb, ones, dstbuf, stage, dinvbuf):
    c = lax.axis_index("c")
    s = lax.axis_index("s")
    on0 = c == 0

    @pl.when(on0)
    def _():
        def fz(i, carry):
            zb[pl.ds(i * 16, 16)] = jnp.zeros((16,), jnp.float32)
            return carry
        lax.fori_loop(0, 64, fz, 0)

        def fo(i, carry):
            ones[pl.ds(i * 16, 16)] = jnp.ones((16,), jnp.float32)
            return carry
        lax.fori_loop(0, 128, fo, 0)
        for j in range(4):
            pltpu.sync_copy(zb, degspm.at[pl.ds(s * 4096 + j * 1024, 1024)])

    plsc.subcore_barrier()

    @pl.when(on0)
    def _():
        base_e = s * EPT

        def blk(j, carry):
            pltpu.sync_copy(dst_hbm.at[pl.ds(base_e + j * 2048, 2048)],
                            dstbuf)
            pltpu.sync_copy(ones, degspm.at[dstbuf], add=True)
            return carry
        lax.fori_loop(0, 25, blk, 0)

    plsc.subcore_barrier()

    @pl.when(on0)
    def _():
        for g in range(5):
            base = s * 3200 + g * 640
            pltpu.sync_copy(degspm.at[pl.ds(base, 640)], stage)

            def sub(k, carry):
                ridx = k * 16 + lax.iota(jnp.int32, 16)
                deg = stage[pl.ds(k * 16, 16)] + 1.0
                y = _fast_rsqrt(deg)
                y = jnp.where(base + ridx < N, y, 0.0)
                dinvbuf[pl.ds(k * 16, 16)] = y
                return carry
            lax.fori_loop(0, 40, sub, 0)
            pltpu.sync_copy(dinvbuf, dinv_hbm.at[pl.ds(base, 640)])


_deg_kernel = functools.partial(
    pl.kernel,
    out_type=jax.ShapeDtypeStruct((ND,), jnp.float32),
    mesh=_MESH,
    scratch_types=[
        pltpu.VMEM_SHARED((NPAD,), jnp.float32),
        pltpu.VMEM((1024,), jnp.float32),
        pltpu.VMEM((2048,), jnp.float32),
        pltpu.VMEM((2048,), jnp.int32),
        pltpu.VMEM((640,), jnp.float32),
        pltpu.VMEM((640,), jnp.float32),
    ],
)(_deg_body)


def _make_agg(F, C, K, B, CAP):
    shift = C.bit_length() - 1
    grp = B // 16
    cc16 = C // 16
    capb = CAP + B + 16

    def body(hs_hbm, src_hbm, dst_hbm, out_hbm, chunk, srcblk, dstblk,
             srcbuf, dstlbuf, idxg0, idxg1, rows0, rows1,
             semg0, semg1, sems0, sems1):
        c = lax.axis_index("c")
        s = lax.axis_index("s")
        base_e = s * EPT

        def do_chunk(ci, carry):
            active = lax.rem(ci, 2) == c
            base_n = ci * C

            @pl.when(active)
            def _():
                pltpu.sync_copy(hs_hbm.at[pl.ds(base_n + s * cc16, cc16)],
                                chunk.at[pl.ds(s * cc16, cc16)])

            plsc.subcore_barrier()

            @pl.when(active)
            def _():
                def pf(i, cy):
                    srcbuf[pl.ds(i * 16, 16)] = jnp.zeros((16,), jnp.int32)
                    dstlbuf[pl.ds(i * 16, 16)] = jnp.full((16,), C, jnp.int32)
                    return cy
                lax.fori_loop(0, capb // 16, pf, 0)

                def fblk(j, cur):
                    pltpu.sync_copy(
                        src_hbm.at[pl.ds(base_e + j * 2048, 2048)], srcblk)
                    pltpu.sync_copy(
                        dst_hbm.at[pl.ds(base_e + j * 2048, 2048)], dstblk)

                    def fin(k, cur2):
                        d = dstblk[pl.ds(k * 16, 16)]
                        sv = srcblk[pl.ds(k * 16, 16)]
                        m = (d >> shift) == ci
                        mi = jnp.where(m, 1, 0)
                        tgt = cur2 + plsc.cumsum(mi) - mi
                        plsc.store_scatter(srcbuf, [tgt], sv, mask=m)
                        plsc.store_scatter(
                            dstlbuf, [tgt], d & (C - 1), mask=m)
                        cnt = jnp.sum(mi)
                        return jnp.minimum(cur2 + cnt, CAP)
                    return lax.fori_loop(0, 128, fin, cur)

                m_cnt = lax.fori_loop(0, 25, fblk, jnp.int32(0))
                nb = (m_cnt + (B - 1)) // B

                def issue_g(j, idxg, rows, semg):
                    pltpu.async_copy(
                        hs_hbm.at[srcbuf.at[pl.ds(j * B, B)]], rows, semg)

                def wait_g(rows, semg):
                    pltpu.make_async_copy(
                        hs_hbm.at[pl.ds(0, B)], rows, semg).wait()

                def fire(j, rows, sems):
                    def g(k, cy):
                        dvec = dstlbuf[pl.ds(j * B + k * 16, 16)]
                        pltpu.async_copy(rows.at[pl.ds(k * 16, 16)],
                                         chunk.at[dvec], sems, add=True)
                        return cy
                    lax.fori_loop(0, grp, g, 0)

                def drain(rows, sems):
                    def g(k, cy):
                        pltpu.make_async_copy(
                            rows.at[pl.ds(0, 16)], chunk.at[pl.ds(0, 16)],
                            sems).wait()
                        return cy
                    lax.fori_loop(0, grp, g, 0)

                def pair(t, cy):
                    j0 = t * 2
                    j1 = j0 + 1
                    has1 = j1 < nb
                    issue_g(j0, idxg0, rows0, semg0)

                    @pl.when(has1)
                    def _():
                        issue_g(j1, idxg1, rows1, semg1)
                    wait_g(rows0, semg0)
                    fire(j0, rows0, sems0)

                    @pl.when(has1)
                    def _():
                        wait_g(rows1, semg1)
                        fire(j1, rows1, sems1)
                    drain(rows0, sems0)

                    @pl.when(has1)
                    def _():
                        drain(rows1, sems1)
                    return cy
                lax.fori_loop(0, (nb + 1) // 2, pair, 0)

            plsc.subcore_barrier()

            @pl.when(active)
            def _():
                pltpu.sync_copy(chunk.at[pl.ds(s * cc16, cc16)],
                                out_hbm.at[pl.ds(base_n + s * cc16, cc16)])

            plsc.subcore_barrier()
            return carry

        lax.fori_loop(0, K, do_chunk, 0)

    return functools.partial(
        pl.kernel,
        out_type=jax.ShapeDtypeStruct((NPAD, F), jnp.float32),
        mesh=_MESH,
        compiler_params=pltpu.CompilerParams(
            needs_layout_passes=False, use_tc_tiling_on_sc=False),
        scratch_types=[
            pltpu.VMEM_SHARED((C + 16, F), jnp.float32),
            pltpu.VMEM((2048,), jnp.int32),
            pltpu.VMEM((2048,), jnp.int32),
            pltpu.VMEM((capb,), jnp.int32),
            pltpu.VMEM((capb,), jnp.int32),
            pltpu.VMEM((B,), jnp.int32),
            pltpu.VMEM((B,), jnp.int32),
            pltpu.VMEM((B, F), jnp.float32),
            pltpu.VMEM((B, F), jnp.float32),
            pltpu.SemaphoreType.DMA,
            pltpu.SemaphoreType.DMA,
            pltpu.SemaphoreType.DMA,
            pltpu.SemaphoreType.DMA,
        ],
    )(body)


_agg80 = _make_agg(80, 8192, 7, 256, 10240)
_agg160 = _make_agg(160, 4096, 13, 128, 6144)
_agg320 = _make_agg(320, 2048, 25, 64, 4096)
_agg624 = _make_agg(624, 1024, 49, 32, 2048)


def _pool_body(h5_hbm, batch_hbm, g_hbm, batchv, rows, acc):
    c = lax.axis_index("c")
    s = lax.axis_index("s")
    w = s * 2 + c

    pltpu.sync_copy(batch_hbm, batchv)

    def lower_bound(tgt):
        def step(i, lohi):
            lo, hi = lohi
            mid = (lo + hi) // 2
            big = batchv[pl.ds(mid, 16)][0] >= tgt
            return (jnp.where(big, lo, mid + 1), jnp.where(big, mid, hi))
        lo, _ = lax.fori_loop(0, 16, step,
                              (jnp.int32(0), jnp.int32(N)))
        return lo

    def graph(gi, carry):
        tgt = w * 16 + gi
        sg = lower_bound(tgt)
        eg = lower_bound(tgt + 1)

        def ia(i, cy):
            acc[pl.ds(i * 16, 16)] = jnp.full((16,), -jnp.inf, jnp.float32)
            return cy
        lax.fori_loop(0, 78, ia, 0)

        nblk = (eg - sg + 31) // 32

        def bl(jb, cy):
            pltpu.sync_copy(h5_hbm.at[pl.ds(sg + jb * 32, 32)], rows)

            def rr(r, cy2):
                @pl.when(sg + jb * 32 + r < eg)
                def _():
                    def colmax(q, cy3):
                        sl = pl.ds(q * 16, 16)
                        acc[sl] = jnp.maximum(acc[sl], rows[r, sl])
                        return cy3
                    lax.fori_loop(0, 78, colmax, 0)
                return cy2
            lax.fori_loop(0, 32, rr, 0)
            return cy
        lax.fori_loop(0, nblk, bl, 0)
        pltpu.sync_copy(acc, g_hbm.at[tgt])
        return carry

    lax.fori_loop(0, 16, graph, 0)


_pool_kernel = functools.partial(
    pl.kernel,
    out_type=jax.ShapeDtypeStruct((G, 1248), jnp.float32),
    mesh=_MESH,
    compiler_params=pltpu.CompilerParams(
        needs_layout_passes=False, use_tc_tiling_on_sc=False),
    scratch_types=[
        pltpu.VMEM((50048,), jnp.int32),
        pltpu.VMEM((32, 1248), jnp.float32),
        pltpu.VMEM((1248,), jnp.float32),
    ],
)(_pool_body)


def _scale_body(x_ref, d_ref, o_ref):
    o_ref[...] = x_ref[...] * d_ref[...]


def _scale(x, dcol):
    f = x.shape[1]
    return pl.pallas_call(
        _scale_body,
        grid=(NPAD // 1024,),
        in_specs=[
            pl.BlockSpec((1024, f), lambda i: (i, 0)),
            pl.BlockSpec((1024, 1), lambda i: (i, 0)),
        ],
        out_specs=pl.BlockSpec((1024, f), lambda i: (i, 0)),
        out_shape=jax.ShapeDtypeStruct((NPAD, f), jnp.float32),
    )(x, dcol)


def _make_layer(fin, fout, post_scale):
    def body(a_ref, d_ref, w_ref, b_ref, o_ref):
        xa = a_ref[...] * d_ref[...]
        y = jnp.dot(xa, w_ref[...], preferred_element_type=jnp.float32,
                    precision=lax.Precision.HIGHEST)
        y = jnp.maximum(y + b_ref[...], 0.0)
        if post_scale:
            y = y * d_ref[...]
        o_ref[...] = y

    def run(agg, dcol, w, b):
        return pl.pallas_call(
            body,
            grid=(NPAD // 512,),
            in_specs=[
                pl.BlockSpec((512, fin), lambda i: (i, 0)),
                pl.BlockSpec((512, 1), lambda i: (i, 0)),
                pl.BlockSpec((fin, fout), lambda i: (0, 0)),
                pl.BlockSpec((1, fout), lambda i: (0, 0)),
            ],
            out_specs=pl.BlockSpec((512, fout), lambda i: (i, 0)),
            out_shape=jax.ShapeDtypeStruct((NPAD, fout), jnp.float32),
        )(agg, dcol, w, b)
    return run


_layer1 = _make_layer(80, 80, True)
_layer2 = _make_layer(80, 160, True)
_layer3 = _make_layer(160, 320, True)
_layer4 = _make_layer(320, 624, True)
_layer5 = _make_layer(624, 1248, False)


def _head_body(g_ref, wg1, bg1, wg2, bg2, wf1, bf1, wf2, bf2, wo, bo, o_ref):
    hp = dict(preferred_element_type=jnp.float32,
              precision=lax.Precision.HIGHEST)
    a = jnp.maximum(jnp.dot(g_ref[...], wg1[...], **hp) + bg1[...], 0.0)
    a = jnp.dot(a, wg2[...], **hp) + bg2[...]
    a = jnp.maximum(jnp.dot(a, wf1[...], **hp) + bf1[...], 0.0)
    a = jnp.maximum(jnp.dot(a, wf2[...], **hp) + bf2[...], 0.0)
    o_ref[...] = jnp.dot(a, wo[...], **hp) + bo[...]


def _head(g, Wg1, bg1, Wg2, bg2, Wf1, bf1, Wf2, bf2, Wo, bo):
    return pl.pallas_call(
        _head_body,
        out_shape=jax.ShapeDtypeStruct((G, 1), jnp.float32),
    )(g, Wg1, bg1[None, :], Wg2, bg2[None, :], Wf1, bf1[None, :],
      Wf2, bf2[None, :], Wo, bo[None, :])


def kernel(x, edge_index, batch, T, P, W1, b1, W2, b2, W3, b3, W4, b4, W5, b5,
           Wg1, bg1, Wg2, bg2, Wf1, bf1, Wf2, bf2, Wo, bo):
    src = jnp.pad(edge_index[0], (0, EPAD - E))
    dst = jnp.pad(edge_index[1], (0, EPAD - E), constant_values=NPAD - 1)
    batch_pad = jnp.pad(batch, (0, 48), constant_values=G)

    dinv = _deg_kernel(dst)
    dcol = jnp.pad(dinv, (0, NPAD - ND))[:, None]

    x_pad = jnp.pad(x, ((0, NPAD - N), (0, 2)))
    w1p = jnp.pad(W1, ((0, 2), (0, 2)))
    b1p = jnp.pad(b1, (0, 2))[None, :]
    w2p = jnp.pad(W2, ((0, 2), (0, 4)))
    b2p = jnp.pad(b2, (0, 4))[None, :]
    w3p = jnp.pad(W3, ((0, 4), (0, 8)))
    b3p = jnp.pad(b3, (0, 8))[None, :]
    w4p = jnp.pad(W4, ((0, 8), (0, 0)))
    b4p = b4[None, :]

    hs = _scale(x_pad, dcol)
    hs = _layer1(_agg80(hs, src, dst), dcol, w1p, b1p)
    hs = _layer2(_agg80(hs, src, dst), dcol, w2p, b2p)
    hs = _layer3(_agg160(hs, src, dst), dcol, w3p, b3p)
    hs = _layer4(_agg320(hs, src, dst), dcol, w4p, b4p)
    h5 = _layer5(_agg624(hs, src, dst), dcol, W5, b5[None, :])

    g = _pool_kernel(h5, batch_pad)
    return _head(g, Wg1, bg1, Wg2, bg2, Wf1, bf1, Wf2, bf2, Wo, bo)
